# baseline (speedup 1.0000x reference)
import functools

import jax
import jax.numpy as jnp
from jax import lax
from jax.experimental import pallas as pl
from jax.experimental.pallas import tpu as pltpu
from jax.experimental.pallas import tpu_sc as plsc

N_PEAKS_ = 64
DIM_ = 128


def _cond_body(x_ref, s_ref, b_ref, o_ref):
    s = s_ref[...][:, :, None]
    b = b_ref[...][:, :, None]
    o_ref[...] = x_ref[...] * s + b


_L = 16


class _Pair:
    def __init__(self, a, b):
        self._cps = (a, b)

    def start(self):
        for c in self._cps:
            c.start()

    def wait(self):
        for c in self._cps:
            c.wait()


def _sc_body(nrows_w, dim, t, x_hbm, ws16_hbm, wb16_hbm, out_hbm,
             in0, in1, in2, sA, sB, bA, bB,
             sin0, sin1, sin2, sout0, sout1, sout2):
    nc = 2
    nw = 32
    wid = lax.axis_index("s") * nc + lax.axis_index("c")
    bufs = (
        (in0, sin0, sout0),
        (in1, sin1, sout1),
        (in2, sin2, sout2),
    )
    wtabs = ((sA, bA), (sB, bB))
    last = nrows_w - 1

    pltpu.sync_copy(ws16_hbm.at[wid], sA)
    pltpu.sync_copy(wb16_hbm.at[wid], bA)
    pltpu.sync_copy(ws16_hbm.at[wid + nw], sB)
    pltpu.sync_copy(wb16_hbm.at[wid + nw], bB)

    hdim = dim // 2

    def in_cp(k, bi):
        ibuf, si, _ = bufs[bi]
        row = wid + nw * k
        return _Pair(
            pltpu.make_async_copy(
                x_hbm.at[row, pl.ds(0, hdim)], ibuf.at[pl.ds(0, hdim)], si),
            pltpu.make_async_copy(
                x_hbm.at[row, pl.ds(hdim, hdim)], ibuf.at[pl.ds(hdim, hdim)], si),
        )

    def out_cp(k, bi):
        ibuf, _, so = bufs[bi]
        row = wid + nw * k
        return _Pair(
            pltpu.make_async_copy(
                ibuf.at[pl.ds(0, hdim)], out_hbm.at[row, pl.ds(0, hdim)], so),
            pltpu.make_async_copy(
                ibuf.at[pl.ds(hdim, hdim)], out_hbm.at[row, pl.ds(hdim, hdim)], so),
        )

    def compute(bi, par):
        ibuf = bufs[bi][0]
        sbuf, bbuf = wtabs[par]

        def do_d(d2, carry2):
            for u in range(2):
                d = d2 * 2 + u
                s = sbuf[pl.ds(d * _L, _L)]
                b = bbuf[pl.ds(d * _L, _L)]
                for tt in range(t // _L):
                    sl = pl.ds(tt * _L, _L)
                    ibuf[d, sl] = ibuf[d, sl] * s + b
            return carry2

        lax.fori_loop(0, dim // 2, do_d, 0)

    def row_step(k, bi, par, prefetch):
        in_cp(k, bi).wait()
        if True:
            compute(bi, par)
        out_cp(k, bi).start()
        if prefetch:
            nbi = (bi + 2) % 3

            @pl.when((k >= 1) & (k <= last - 2))
            def _():
                out_cp(k - 1, nbi).wait()

            @pl.when(k <= last - 2)
            def _():
                in_cp(k + 2, nbi).start()

    in_cp(0, 0).start()
    in_cp(1, 1).start()

    def body(j, carry):
        a = 6 * j
        for u in range(6):
            row_step(a + u, u % 3, u % 2, True)
        return carry

    lax.fori_loop(0, nrows_w // 6, body, 0)
    for u in range(4):
        k = (nrows_w // 6) * 6 + u
        row_step(k, k % 3, k % 2, k <= last - 2)
    out_cp(last - 2, (last - 2) % 3).wait()
    out_cp(last - 1, (last - 1) % 3).wait()
    out_cp(last, last % 3).wait()


def _sc_kernel(x, W_scale, W_bias):
    rows, dim, t = x.shape
    nw = 32
    nrows_w = rows // nw
    assert nrows_w == N_PEAKS_
    ws16 = jnp.repeat(W_scale.reshape(N_PEAKS_, dim, 1), _L, axis=2).reshape(
        N_PEAKS_, dim * _L)
    wb16 = jnp.repeat(W_bias.reshape(N_PEAKS_, dim, 1), _L, axis=2).reshape(
        N_PEAKS_, dim * _L)
    mesh = plsc.VectorSubcoreMesh(core_axis_name="c", subcore_axis_name="s")
    f = pl.kernel(
        functools.partial(_sc_body, nrows_w, dim, t),
        out_type=jax.ShapeDtypeStruct(x.shape, x.dtype),
        mesh=mesh,
        scratch_types=(
            [pltpu.VMEM((dim, t), jnp.float32)] * 3
            + [pltpu.VMEM((dim * _L,), jnp.float32)] * 4
            + [pltpu.SemaphoreType.DMA] * 6
        ),
    )
    return f(x, ws16, wb16)


def kernel(x, queries, W_scale, W_bias):
    del queries
    return _sc_kernel(x, W_scale, W_bias)
    rows, dim, t = x.shape
    R = 64
    grid = (rows // R,)

    out = pl.pallas_call(
        _cond_body,
        grid=grid,
        in_specs=[
            pl.BlockSpec((R, dim, t), lambda i: (i, 0, 0)),
            pl.BlockSpec((N_PEAKS_, dim), lambda i: (0, 0)),
            pl.BlockSpec((N_PEAKS_, dim), lambda i: (0, 0)),
        ],
        out_specs=pl.BlockSpec((R, dim, t), lambda i: (i, 0, 0)),
        out_shape=jax.ShapeDtypeStruct(x.shape, x.dtype),
        compiler_params=pltpu.CompilerParams(
            dimension_semantics=("parallel",),
        ),
    )(x, W_scale, W_bias)
    return out


# speedup vs baseline: 1.0031x; 1.0031x over previous
import functools

import jax
import jax.numpy as jnp
from jax import lax
from jax.experimental import pallas as pl
from jax.experimental.pallas import tpu as pltpu
from jax.experimental.pallas import tpu_sc as plsc

N_PEAKS_ = 64

_L = 16


def _sc_body(nrows_w, dim, t, x_hbm, ws16_hbm, wb16_hbm, out_hbm,
             in0, in1, in2, sA, sB, bA, bB,
             sin0, sin1, sin2, sout0, sout1, sout2):
    nc = 2
    nw = 32
    wid = lax.axis_index("s") * nc + lax.axis_index("c")
    bufs = (
        (in0, sin0, sout0),
        (in1, sin1, sout1),
        (in2, sin2, sout2),
    )
    wtabs = ((sA, bA), (sB, bB))
    last = nrows_w - 1

    pltpu.sync_copy(ws16_hbm.at[wid], sA)
    pltpu.sync_copy(wb16_hbm.at[wid], bA)
    pltpu.sync_copy(ws16_hbm.at[wid + nw], sB)
    pltpu.sync_copy(wb16_hbm.at[wid + nw], bB)

    def in_cp(k, bi):
        ibuf, si, _ = bufs[bi]
        return pltpu.make_async_copy(x_hbm.at[wid + nw * k], ibuf, si)

    def out_cp(k, bi):
        ibuf, _, so = bufs[bi]
        return pltpu.make_async_copy(ibuf, out_hbm.at[wid + nw * k], so)

    def compute(bi, par):
        ibuf = bufs[bi][0]
        sbuf, bbuf = wtabs[par]

        def do_d(d2, carry2):
            for u in range(2):
                d = d2 * 2 + u
                s = sbuf[pl.ds(d * _L, _L)]
                b = bbuf[pl.ds(d * _L, _L)]
                for tt in range(t // _L):
                    sl = pl.ds(tt * _L, _L)
                    ibuf[d, sl] = ibuf[d, sl] * s + b
            return carry2

        lax.fori_loop(0, dim // 2, do_d, 0)

    def row_step(k, bi, par, prefetch):
        in_cp(k, bi).wait()
        compute(bi, par)
        out_cp(k, bi).start()
        if prefetch:
            nbi = (bi + 2) % 3

            @pl.when((k >= 1) & (k <= last - 2))
            def _():
                out_cp(k - 1, nbi).wait()

            @pl.when(k <= last - 2)
            def _():
                in_cp(k + 2, nbi).start()

    in_cp(0, 0).start()
    in_cp(1, 1).start()

    def body(j, carry):
        a = 6 * j
        for u in range(6):
            row_step(a + u, u % 3, u % 2, True)
        return carry

    lax.fori_loop(0, nrows_w // 6, body, 0)
    for u in range(4):
        k = (nrows_w // 6) * 6 + u
        row_step(k, k % 3, k % 2, k <= last - 2)
    out_cp(last - 2, (last - 2) % 3).wait()
    out_cp(last - 1, (last - 1) % 3).wait()
    out_cp(last, last % 3).wait()


def _sc_kernel(x, W_scale, W_bias):
    rows, dim, t = x.shape
    nw = 32
    nrows_w = rows // nw
    assert N_PEAKS_ == 2 * nw and rows % nw == 0
    ws16 = jnp.repeat(W_scale.reshape(N_PEAKS_, dim, 1), _L, axis=2).reshape(
        N_PEAKS_, dim * _L)
    wb16 = jnp.repeat(W_bias.reshape(N_PEAKS_, dim, 1), _L, axis=2).reshape(
        N_PEAKS_, dim * _L)
    mesh = plsc.VectorSubcoreMesh(core_axis_name="c", subcore_axis_name="s")
    f = pl.kernel(
        functools.partial(_sc_body, nrows_w, dim, t),
        out_type=jax.ShapeDtypeStruct(x.shape, x.dtype),
        mesh=mesh,
        scratch_types=(
            [pltpu.VMEM((dim, t), jnp.float32)] * 3
            + [pltpu.VMEM((dim * _L,), jnp.float32)] * 4
            + [pltpu.SemaphoreType.DMA] * 6
        ),
    )
    return f(x, ws16, wb16)


def kernel(x, queries, W_scale, W_bias):
    del queries
    return _sc_kernel(x, W_scale, W_bias)
